# baseline (speedup 1.0000x reference)
import functools

import jax
import jax.numpy as jnp
from jax import lax
from jax.experimental import pallas as pl
from jax.experimental.pallas import tpu as pltpu
from jax.experimental.pallas import tpu_sc as plsc

N_CORES = 2
N_SUBCORES = 16
LANES = 16
CHUNK = 128
PIPE = 5


@functools.lru_cache(maxsize=None)
def _make_spmm(n_pad, d_blk, e_pad):
    ept = e_pad // N_SUBCORES
    nch = ept // CHUNK
    rpt = n_pad // N_SUBCORES
    n_sub = d_blk // LANES

    mesh = plsc.VectorSubcoreMesh(
        core_axis_name="c", subcore_axis_name="s",
        num_cores=N_CORES, num_subcores=N_SUBCORES)

    NB = PIPE
    assert nch % NB == 0

    @functools.partial(
        pl.kernel,
        out_type=jax.ShapeDtypeStruct((N_CORES * n_pad, d_blk), jnp.float32),
        mesh=mesh,
        compiler_params=pltpu.CompilerParams(use_tc_tiling_on_sc=False),
        scratch_types=[
            pltpu.VMEM((ept,), jnp.int32),
            pltpu.VMEM((ept,), jnp.int32),
            [pltpu.VMEM((CHUNK,), jnp.float32)] * NB,
            [pltpu.VMEM((CHUNK,), jnp.int32)] * NB,
            [pltpu.VMEM((CHUNK, d_blk), jnp.float32)] * NB,
            [pltpu.SemaphoreType.DMA] * NB,
            [pltpu.SemaphoreType.DMA] * NB,
            [pltpu.SemaphoreType.DMA] * NB,
            pltpu.VMEM_SHARED((n_pad, d_blk), jnp.float32),
        ],
    )
    def spmm(h, dstp, srcp2, wp, out, dstv, srcv, wc,
             dstc, rowb, gsem, ssem, wsem, hacc):
        c = lax.axis_index("c")
        s = lax.axis_index("s")
        t0 = s * ept

        pltpu.sync_copy(dstp.at[pl.ds(t0, ept)], dstv)
        pltpu.sync_copy(srcp2.at[pl.ds(c * e_pad + t0, ept)], srcv)

        zeros16 = jnp.zeros((LANES,), jnp.float32)

        def zrow(i, carry):
            for j in range(n_sub):
                rowb[0][i, pl.ds(j * LANES, LANES)] = zeros16
            return carry

        lax.fori_loop(0, CHUNK, zrow, 0)
        r0 = s * rpt
        full, rem = divmod(rpt, CHUNK)
        for q in range(full):
            pltpu.sync_copy(rowb[0], hacc.at[pl.ds(r0 + q * CHUNK, CHUNK)])
        if rem:
            pltpu.sync_copy(rowb[0].at[pl.ds(0, rem)],
                            hacc.at[pl.ds(r0 + full * CHUNK, rem)])
        plsc.subcore_barrier()

        cbase = c * n_pad

        def stage_dst(loc, dstc):
            for j in range(CHUNK // LANES):
                dstc[pl.ds(j * LANES, LANES)] = dstv[pl.ds(loc + j * LANES, LANES)]

        def issue(k, b):
            loc = k * CHUNK
            stage_dst(loc, dstc[b])
            pltpu.async_copy(wp.at[pl.ds(t0 + loc, CHUNK)], wc[b], wsem[b])
            pltpu.async_copy(h.at[srcv.at[pl.ds(loc, CHUNK)]], rowb[b], gsem[b])

        def mul(b):
            def body(g, carry):
                gbase = g * LANES
                wvec = wc[b][pl.ds(gbase, LANES)]
                for l in range(LANES):
                    wb = wvec[l]
                    for j in range(n_sub):
                        sl2 = pl.ds(j * LANES, LANES)
                        rowb[b][gbase + l, sl2] = rowb[b][gbase + l, sl2] * wb
                return carry

            lax.fori_loop(0, CHUNK // LANES, body, 0)

        for b in range(NB - 1):
            issue(b, b)

        def outer(kq, carry):
            for b in range(NB):
                k = kq * NB + b
                bp = (b + NB - 1) % NB
                pltpu.make_async_copy(
                    h.at[srcv.at[pl.ds(k * CHUNK, CHUNK)]],
                    rowb[b], gsem[b]).wait()
                pltpu.make_async_copy(
                    wp.at[pl.ds(t0 + k * CHUNK, CHUNK)], wc[b], wsem[b]).wait()
                mul(b)

                if b == 0:
                    @pl.when(kq > 0)
                    def _():
                        pltpu.make_async_copy(
                            rowb[bp], hacc.at[dstc[bp]], ssem[bp]).wait()

                    issue(k + NB - 1, bp)
                else:
                    @pl.when(k + NB - 1 < nch)
                    def _():
                        pltpu.make_async_copy(
                            rowb[bp], hacc.at[dstc[bp]], ssem[bp]).wait()
                        issue(k + NB - 1, bp)

                pltpu.async_copy(rowb[b], hacc.at[dstc[b]], ssem[b], add=True)
            return carry

        lax.fori_loop(0, nch // NB, outer, 0)
        for b in range(NB):
            pltpu.make_async_copy(rowb[b], hacc.at[dstc[b]], ssem[b]).wait()
        plsc.subcore_barrier()

        pltpu.sync_copy(hacc.at[pl.ds(r0, rpt)],
                        out.at[pl.ds(cbase + r0, rpt)])

    return spmm


def kernel(x, edge_index, edge_weight, n_layers):
    n, d = x.shape
    e = edge_index.shape[1]
    d_blk = d // N_CORES
    rpt = -(-n // N_SUBCORES)
    rpt = ((rpt + 7) // 8) * 8
    n_pad = rpt * N_SUBCORES

    grp = N_SUBCORES * CHUNK * PIPE
    e_pad = ((e + grp - 1) // grp) * grp
    dst = edge_index[0].astype(jnp.int32)
    src = edge_index[1].astype(jnp.int32)
    w = edge_weight.astype(jnp.float32)
    if e_pad != e:
        pad = e_pad - e
        dst = jnp.pad(dst, (0, pad))
        src = jnp.pad(src, (0, pad))
        w = jnp.pad(w, (0, pad))
    src2 = jnp.concatenate([src, src + n_pad])

    h0 = jnp.concatenate(
        [jnp.pad(x[:, i * d_blk:(i + 1) * d_blk], ((0, n_pad - n), (0, 0)))
         for i in range(N_CORES)], axis=0)

    spmm = _make_spmm(n_pad, d_blk, e_pad)

    def body(_, h):
        return spmm(h, dst, src2, w)

    hout = lax.fori_loop(0, n_layers, body, h0)
    return jnp.concatenate([hout[i * n_pad:i * n_pad + n]
                            for i in range(N_CORES)], axis=1)


# speedup vs baseline: 1.6223x; 1.6223x over previous
import functools

import jax
import jax.numpy as jnp
from jax import lax
from jax.experimental import pallas as pl
from jax.experimental.pallas import tpu as pltpu
from jax.experimental.pallas import tpu_sc as plsc

N_CORES = 2
N_SUBCORES = 16
LANES = 16
CHUNK = 128
PIPE = 3


@functools.lru_cache(maxsize=None)
def _make_spmm(n_pad, d_blk, e_pad):
    ept = e_pad // N_SUBCORES
    nch = ept // CHUNK
    rpt = n_pad // N_SUBCORES
    n_sub = d_blk // LANES

    mesh = plsc.VectorSubcoreMesh(
        core_axis_name="c", subcore_axis_name="s",
        num_cores=N_CORES, num_subcores=N_SUBCORES)

    NB = PIPE
    assert nch % NB == 0

    @functools.partial(
        pl.kernel,
        out_type=jax.ShapeDtypeStruct((N_CORES * n_pad, d_blk), jnp.float32),
        mesh=mesh,
        compiler_params=pltpu.CompilerParams(use_tc_tiling_on_sc=False),
        scratch_types=[
            pltpu.VMEM((ept,), jnp.int32),
            pltpu.VMEM((ept,), jnp.int32),
            pltpu.VMEM((ept,), jnp.float32),
            [pltpu.VMEM((CHUNK,), jnp.int32)] * NB,
            [pltpu.VMEM((CHUNK,), jnp.int32)] * NB,
            [pltpu.VMEM((CHUNK, d_blk), jnp.float32)] * NB,
            [pltpu.SemaphoreType.DMA] * NB,
            [pltpu.SemaphoreType.DMA] * NB,
            pltpu.VMEM_SHARED((n_pad, d_blk), jnp.float32),
        ],
    )
    def spmm(h, dstp, srcp, wp, out, dstv, srcv, wv,
             srcc, dstc, rowb, gsem, ssem, hacc):
        c = lax.axis_index("c")
        s = lax.axis_index("s")
        t0 = s * ept

        pltpu.sync_copy(dstp.at[pl.ds(t0, ept)], dstv)
        pltpu.sync_copy(srcp.at[pl.ds(t0, ept)], srcv)
        pltpu.sync_copy(wp.at[pl.ds(t0, ept)], wv)

        zeros16 = jnp.zeros((LANES,), jnp.float32)

        def zrow(i, carry):
            for j in range(n_sub):
                rowb[0][i, pl.ds(j * LANES, LANES)] = zeros16
            return carry

        lax.fori_loop(0, CHUNK, zrow, 0)
        r0 = s * rpt
        full, rem = divmod(rpt, CHUNK)
        for q in range(full):
            pltpu.sync_copy(rowb[0], hacc.at[pl.ds(r0 + q * CHUNK, CHUNK)])
        if rem:
            pltpu.sync_copy(rowb[0].at[pl.ds(0, rem)],
                            hacc.at[pl.ds(r0 + full * CHUNK, rem)])
        plsc.subcore_barrier()

        cbase = c * n_pad

        def stage(loc, srcc, dstc):
            for j in range(CHUNK // LANES):
                sl = pl.ds(j * LANES, LANES)
                slv = pl.ds(loc + j * LANES, LANES)
                srcc[sl] = srcv[slv] + cbase
                dstc[sl] = dstv[slv]

        def mul(loc, rowb):
            def body(g, carry):
                gbase = g * LANES
                wvec = wv[pl.ds(loc + gbase, LANES)]
                for l in range(LANES):
                    wb = wvec[l]
                    for j in range(n_sub):
                        sl2 = pl.ds(j * LANES, LANES)
                        rowb[gbase + l, sl2] = rowb[gbase + l, sl2] * wb
                return carry

            lax.fori_loop(0, CHUNK // LANES, body, 0)

        for b in range(NB - 1):
            stage(b * CHUNK, srcc[b], dstc[b])
            pltpu.async_copy(h.at[srcc[b]], rowb[b], gsem[b])

        def outer(kq, carry):
            for b in range(NB):
                k = kq * NB + b
                bp = (b + NB - 1) % NB
                pltpu.make_async_copy(h.at[srcc[b]], rowb[b], gsem[b]).wait()
                mul(k * CHUNK, rowb[b])

                def refill():
                    stage((k + NB - 1) * CHUNK, srcc[bp], dstc[bp])
                    pltpu.async_copy(h.at[srcc[bp]], rowb[bp], gsem[bp])

                if b == 0:
                    @pl.when(kq > 0)
                    def _():
                        pltpu.make_async_copy(
                            rowb[bp], hacc.at[dstc[bp]], ssem[bp]).wait()

                    refill()
                else:
                    @pl.when(k + NB - 1 < nch)
                    def _():
                        pltpu.make_async_copy(
                            rowb[bp], hacc.at[dstc[bp]], ssem[bp]).wait()
                        refill()

                pltpu.async_copy(rowb[b], hacc.at[dstc[b]], ssem[b], add=True)
            return carry

        lax.fori_loop(0, nch // NB, outer, 0)
        for b in range(NB):
            pltpu.make_async_copy(rowb[b], hacc.at[dstc[b]], ssem[b]).wait()
        plsc.subcore_barrier()

        pltpu.sync_copy(hacc.at[pl.ds(r0, rpt)],
                        out.at[pl.ds(cbase + r0, rpt)])

    return spmm


def kernel(x, edge_index, edge_weight, n_layers):
    n, d = x.shape
    e = edge_index.shape[1]
    d_blk = d // N_CORES
    rpt = -(-n // N_SUBCORES)
    rpt = ((rpt + 7) // 8) * 8
    n_pad = rpt * N_SUBCORES

    grp = N_SUBCORES * CHUNK * PIPE
    e_pad = ((e + grp - 1) // grp) * grp
    dst = edge_index[0].astype(jnp.int32)
    src = edge_index[1].astype(jnp.int32)
    w = edge_weight.astype(jnp.float32)
    if e_pad != e:
        pad = e_pad - e
        dst = jnp.pad(dst, (0, pad))
        src = jnp.pad(src, (0, pad))
        w = jnp.pad(w, (0, pad))

    h0 = jnp.concatenate(
        [jnp.pad(x[:, i * d_blk:(i + 1) * d_blk], ((0, n_pad - n), (0, 0)))
         for i in range(N_CORES)], axis=0)

    spmm = _make_spmm(n_pad, d_blk, e_pad)

    def body(_, h):
        return spmm(h, dst, src, w)

    hout = lax.fori_loop(0, n_layers, body, h0)
    return jnp.concatenate([hout[i * n_pad:i * n_pad + n]
                            for i in range(N_CORES)], axis=1)


# speedup vs baseline: 1.6416x; 1.0119x over previous
import functools

import jax
import jax.numpy as jnp
from jax import lax
from jax.experimental import pallas as pl
from jax.experimental.pallas import tpu as pltpu
from jax.experimental.pallas import tpu_sc as plsc

N_CORES = 2
N_SUBCORES = 16
LANES = 16
CHUNK = 128
PIPE = 3


@functools.lru_cache(maxsize=None)
def _make_spmm(n_pad, d_blk, e_pad):
    ept = e_pad // N_SUBCORES
    nch = ept // CHUNK
    rpt = n_pad // N_SUBCORES
    n_sub = d_blk // LANES

    mesh = plsc.VectorSubcoreMesh(
        core_axis_name="c", subcore_axis_name="s",
        num_cores=N_CORES, num_subcores=N_SUBCORES)

    NB = PIPE
    assert nch % NB == 0

    @functools.partial(
        pl.kernel,
        out_type=jax.ShapeDtypeStruct((N_CORES * n_pad, d_blk), jnp.float32),
        mesh=mesh,
        compiler_params=pltpu.CompilerParams(use_tc_tiling_on_sc=False),
        scratch_types=[
            pltpu.VMEM((ept,), jnp.int32),
            pltpu.VMEM((ept,), jnp.int32),
            pltpu.VMEM((ept,), jnp.float32),
            [pltpu.VMEM((CHUNK,), jnp.int32)] * NB,
            [pltpu.VMEM((CHUNK,), jnp.int32)] * NB,
            [pltpu.VMEM((CHUNK, d_blk), jnp.float32)] * NB,
            [pltpu.SemaphoreType.DMA] * NB,
            [pltpu.SemaphoreType.DMA] * NB,
            [pltpu.SemaphoreType.DMA] * 3,
            pltpu.VMEM_SHARED((n_pad, d_blk), jnp.float32),
        ],
    )
    def spmm(h, dstp, srcp, wp, out, dstv, srcv, wv,
             srcc, dstc, rowb, gsem, ssem, esem, hacc):
        c = lax.axis_index("c")
        s = lax.axis_index("s")
        t0 = s * ept

        pltpu.async_copy(dstp.at[pl.ds(t0, ept)], dstv, esem[0])
        pltpu.async_copy(srcp.at[pl.ds(t0, ept)], srcv, esem[1])
        pltpu.async_copy(wp.at[pl.ds(t0, ept)], wv, esem[2])

        zeros16 = jnp.zeros((LANES,), jnp.float32)

        def zrow(i, carry):
            for j in range(n_sub):
                rowb[0][i, pl.ds(j * LANES, LANES)] = zeros16
            return carry

        lax.fori_loop(0, CHUNK, zrow, 0)
        r0 = s * rpt
        full, rem = divmod(rpt, CHUNK)
        for q in range(full):
            pltpu.sync_copy(rowb[0], hacc.at[pl.ds(r0 + q * CHUNK, CHUNK)])
        if rem:
            pltpu.sync_copy(rowb[0].at[pl.ds(0, rem)],
                            hacc.at[pl.ds(r0 + full * CHUNK, rem)])
        pltpu.make_async_copy(dstp.at[pl.ds(t0, ept)], dstv, esem[0]).wait()
        pltpu.make_async_copy(srcp.at[pl.ds(t0, ept)], srcv, esem[1]).wait()
        pltpu.make_async_copy(wp.at[pl.ds(t0, ept)], wv, esem[2]).wait()
        plsc.subcore_barrier()

        cbase = c * n_pad

        def stage(loc, srcc, dstc):
            for j in range(CHUNK // LANES):
                sl = pl.ds(j * LANES, LANES)
                slv = pl.ds(loc + j * LANES, LANES)
                srcc[sl] = srcv[slv] + cbase
                dstc[sl] = dstv[slv]

        def mul(loc, rowb):
            def body(g, carry):
                gbase = g * LANES
                wvec = wv[pl.ds(loc + gbase, LANES)]
                for l in range(LANES):
                    wb = wvec[l]
                    for j in range(n_sub):
                        sl2 = pl.ds(j * LANES, LANES)
                        rowb[gbase + l, sl2] = rowb[gbase + l, sl2] * wb
                return carry

            lax.fori_loop(0, CHUNK // LANES, body, 0)

        for b in range(NB - 1):
            stage(b * CHUNK, srcc[b], dstc[b])
            pltpu.async_copy(h.at[srcc[b]], rowb[b], gsem[b])

        def outer(kq, carry):
            for b in range(NB):
                k = kq * NB + b
                bp = (b + NB - 1) % NB
                pltpu.make_async_copy(h.at[srcc[b]], rowb[b], gsem[b]).wait()
                mul(k * CHUNK, rowb[b])

                def refill():
                    stage((k + NB - 1) * CHUNK, srcc[bp], dstc[bp])
                    pltpu.async_copy(h.at[srcc[bp]], rowb[bp], gsem[bp])

                if b == 0:
                    @pl.when(kq > 0)
                    def _():
                        pltpu.make_async_copy(
                            rowb[bp], hacc.at[dstc[bp]], ssem[bp]).wait()

                    refill()
                else:
                    @pl.when(k + NB - 1 < nch)
                    def _():
                        pltpu.make_async_copy(
                            rowb[bp], hacc.at[dstc[bp]], ssem[bp]).wait()
                        refill()

                pltpu.async_copy(rowb[b], hacc.at[dstc[b]], ssem[b], add=True)
            return carry

        lax.fori_loop(0, nch // NB, outer, 0)
        for b in range(NB):
            pltpu.make_async_copy(rowb[b], hacc.at[dstc[b]], ssem[b]).wait()
        plsc.subcore_barrier()

        pltpu.sync_copy(hacc.at[pl.ds(r0, rpt)],
                        out.at[pl.ds(cbase + r0, rpt)])

    return spmm


def kernel(x, edge_index, edge_weight, n_layers):
    n, d = x.shape
    e = edge_index.shape[1]
    d_blk = d // N_CORES
    rpt = -(-n // N_SUBCORES)
    rpt = ((rpt + 7) // 8) * 8
    n_pad = rpt * N_SUBCORES

    grp = N_SUBCORES * CHUNK * PIPE
    e_pad = ((e + grp - 1) // grp) * grp
    dst = edge_index[0].astype(jnp.int32)
    src = edge_index[1].astype(jnp.int32)
    w = edge_weight.astype(jnp.float32)
    if e_pad != e:
        pad = e_pad - e
        dst = jnp.pad(dst, (0, pad))
        src = jnp.pad(src, (0, pad))
        w = jnp.pad(w, (0, pad))

    h0 = jnp.concatenate(
        [jnp.pad(x[:, i * d_blk:(i + 1) * d_blk], ((0, n_pad - n), (0, 0)))
         for i in range(N_CORES)], axis=0)

    spmm = _make_spmm(n_pad, d_blk, e_pad)

    def body(_, h):
        return spmm(h, dst, src, w)

    hout = lax.fori_loop(0, n_layers, body, h0)
    return jnp.concatenate([hout[i * n_pad:i * n_pad + n]
                            for i in range(N_CORES)], axis=1)
